# Initial kernel scaffold; baseline (speedup 1.0000x reference)
import jax
import jax.numpy as jnp
from jax.experimental import pallas as pl


def kernel(x, edge_index, W_self1, W_neigh1, b1, W_self2, W_neigh2, b2, W_fc1, b_fc1, W_fc2, b_fc2):
    raise NotImplementedError("write your pallas kernel here")


# speedup vs baseline: 2.9235x; 2.9235x over previous
import jax
import jax.numpy as jnp
from jax import lax
from jax.experimental import pallas as pl
from jax.experimental.pallas import tpu as pltpu
from jax.experimental.pallas import tpu_sc as plsc

N = 10000
NP = 10240
E = 160000
DC = 128
NC = 2
NS = 16
C = 80
EPT = E // NS
ITERS = EPT // C
C2 = 40
EPT2 = E // (NC * NS)
ITERS2 = EPT2 // C2
RPT = NP // NS
ZR = 64


def _fill2d(ref, val):
    rows, cols = ref.shape
    nv = cols // 16
    vec = jnp.full((16,), val, ref.dtype)

    def body(i, carry):
        r = i // nv
        j = i - r * nv
        ref[r, pl.ds(j * 16, 16)] = vec
        return carry

    lax.fori_loop(0, rows * nv, body, 0)


def _make_sc_agg(q_chunks, with_deg):
    mesh = plsc.VectorSubcoreMesh(core_axis_name="c", subcore_axis_name="s")
    out_type = [jax.ShapeDtypeStruct((q_chunks * NP, DC), jnp.float32)]
    if with_deg:
        out_type.append(jax.ShapeDtypeStruct((NC * NP, DC), jnp.float32))
    scratch = [
        pltpu.VMEM((C,), jnp.int32),
        pltpu.VMEM((C,), jnp.int32),
        pltpu.VMEM((C, DC), jnp.float32),
        pltpu.VMEM((ZR, DC), jnp.float32),
        pltpu.VMEM((C2,), jnp.int32),
        pltpu.VMEM((C2, DC), jnp.float32),
        pltpu.MemorySpace.VMEM_SHARED((NP, DC), jnp.float32),
        pltpu.SemaphoreType.DMA,
    ]

    def body(table, src, dst, agg_out, *rest):
        if with_deg:
            deg_out = rest[0]
            rest = rest[1:]
        src_idx, dst_idx, rows, zbuf, dst_idx2, ones, acc, sem = rest
        c = lax.axis_index("c")
        s = lax.axis_index("s")
        r0 = s * RPT

        _fill2d(zbuf, 0.0)
        if with_deg:
            _fill2d(ones, 1.0)

        n_passes = q_chunks // NC
        for p in range(n_passes):
            qn = (p * NC + c) * NP

            for k in range(RPT // ZR):
                pltpu.sync_copy(zbuf, acc.at[pl.ds(r0 + k * ZR, ZR)])
            plsc.subcore_barrier()

            def edge_body(i, carry):
                base = s * EPT + i * C
                pltpu.sync_copy(src.at[pl.ds(base, C)], src_idx)
                pltpu.sync_copy(dst.at[pl.ds(base, C)], dst_idx)
                for j in range(C // 16):
                    sl = pl.ds(j * 16, 16)
                    src_idx[sl] = src_idx[sl] + qn
                pltpu.async_copy(table.at[src_idx], rows, sem).wait()
                pltpu.sync_copy(rows, acc.at[dst_idx], add=True)
                return carry

            lax.fori_loop(0, ITERS, edge_body, 0)
            plsc.subcore_barrier()

            pltpu.sync_copy(acc.at[pl.ds(r0, RPT)],
                            agg_out.at[pl.ds(qn + r0, RPT)])
            if p + 1 < n_passes or with_deg:
                plsc.subcore_barrier()

        if with_deg:
            for k in range(RPT // ZR):
                pltpu.sync_copy(zbuf, acc.at[pl.ds(r0 + k * ZR, ZR)])
            plsc.subcore_barrier()

            def deg_body(i, carry):
                base = c * (E // NC) + s * EPT2 + i * C2
                pltpu.sync_copy(dst.at[pl.ds(base, C2)], dst_idx2)
                pltpu.sync_copy(ones, acc.at[dst_idx2], add=True)
                return carry

            lax.fori_loop(0, ITERS2, deg_body, 0)
            plsc.subcore_barrier()
            pltpu.sync_copy(acc.at[pl.ds(r0, RPT)],
                            deg_out.at[pl.ds(c * NP + r0, RPT)])

    return pl.kernel(body, out_type=out_type, mesh=mesh, scratch_types=scratch)


BN = 640


def _k1_body(x_ref, a0_ref, a1_ref, d0_ref, d1_ref, ws_ref, wn_ref, b_ref,
             o_ref):
    agg = jnp.concatenate([a0_ref[...], a1_ref[...]], axis=1)
    deg = d0_ref[...][:, 0:1] + d1_ref[...][:, 0:1]
    recip = 1.0 / jnp.maximum(deg, 1.0)
    acc = jnp.dot(x_ref[...], ws_ref[...], preferred_element_type=jnp.float32)
    acc = acc + jnp.dot(agg, wn_ref[...], preferred_element_type=jnp.float32) * recip
    o_ref[...] = jnp.maximum(acc + b_ref[...], 0.0)


def _layer1(x, agg1c, deg2, w_self, w_neigh, b):
    h = x.shape[1]
    nb = NP // BN
    return pl.pallas_call(
        _k1_body,
        grid=(nb, 4),
        in_specs=[
            pl.BlockSpec((BN, h), lambda i, q: (i, 0)),
            pl.BlockSpec((BN, DC), lambda i, q: (i, 0)),
            pl.BlockSpec((BN, DC), lambda i, q: (nb + i, 0)),
            pl.BlockSpec((BN, DC), lambda i, q: (i, 0)),
            pl.BlockSpec((BN, DC), lambda i, q: (nb + i, 0)),
            pl.BlockSpec((h, DC), lambda i, q: (0, q)),
            pl.BlockSpec((h, DC), lambda i, q: (0, q)),
            pl.BlockSpec((1, DC), lambda i, q: (0, q)),
        ],
        out_specs=pl.BlockSpec((BN, DC), lambda i, q: (q * nb + i, 0)),
        out_shape=jax.ShapeDtypeStruct((4 * NP, DC), jnp.float32),
    )(x, agg1c, agg1c, deg2, deg2, w_self, w_neigh, b)


def _k2_body(h0, h1, h2, h3, a0, a1, a2, a3, d0_ref, d1_ref, ws_ref, wn_ref,
             b2_ref, wf1_ref, bf1_ref, wf2_ref, bf2_ref, o_ref):
    h = jnp.concatenate([h0[...], h1[...], h2[...], h3[...]], axis=1)
    agg = jnp.concatenate([a0[...], a1[...], a2[...], a3[...]], axis=1)
    deg = d0_ref[...][:, 0:1] + d1_ref[...][:, 0:1]
    recip = 1.0 / jnp.maximum(deg, 1.0)
    acc = jnp.dot(h, ws_ref[...], preferred_element_type=jnp.float32)
    acc = acc + jnp.dot(agg, wn_ref[...], preferred_element_type=jnp.float32) * recip
    h2v = jnp.maximum(acc + b2_ref[...], 0.0)
    z = jnp.maximum(
        jnp.dot(h2v, wf1_ref[...], preferred_element_type=jnp.float32) + bf1_ref[...],
        0.0)
    o_ref[...] = (jnp.dot(z, wf2_ref[...], preferred_element_type=jnp.float32)
                  + bf2_ref[...])


def _layer2_head(h1c, agg2c, deg2, w_self, w_neigh, b2, w_fc1, b_fc1, w_fc2,
                 b_fc2):
    nb = NP // BN
    hh = w_self.shape[0]
    chunk_spec = lambda q: pl.BlockSpec((BN, DC), lambda i, q=q: (q * nb + i, 0))
    return pl.pallas_call(
        _k2_body,
        grid=(nb,),
        in_specs=(
            [chunk_spec(q) for q in range(4)]
            + [chunk_spec(q) for q in range(4)]
            + [
                pl.BlockSpec((BN, DC), lambda i: (i, 0)),
                pl.BlockSpec((BN, DC), lambda i: (nb + i, 0)),
                pl.BlockSpec((hh, hh), lambda i: (0, 0)),
                pl.BlockSpec((hh, hh), lambda i: (0, 0)),
                pl.BlockSpec((1, hh), lambda i: (0, 0)),
                pl.BlockSpec((hh, hh), lambda i: (0, 0)),
                pl.BlockSpec((1, hh), lambda i: (0, 0)),
                pl.BlockSpec((hh, 1), lambda i: (0, 0)),
                pl.BlockSpec((1, 1), lambda i: (0, 0)),
            ]
        ),
        out_specs=pl.BlockSpec((BN, 1), lambda i: (i, 0)),
        out_shape=jax.ShapeDtypeStruct((NP, 1), jnp.float32),
    )(*([h1c] * 4), *([agg2c] * 4), deg2, deg2, w_self, w_neigh, b2, w_fc1,
      b_fc1, w_fc2, b_fc2)


def kernel(x, edge_index, W_self1, W_neigh1, b1, W_self2, W_neigh2, b2,
           W_fc1, b_fc1, W_fc2, b_fc2):
    src = edge_index[0]
    dst = edge_index[1]
    xp = jnp.pad(x, ((0, NP - N), (0, 0)))
    xc = xp.reshape(NP, 2, DC).transpose(1, 0, 2).reshape(2 * NP, DC)

    agg1c, deg2 = _make_sc_agg(2, True)(xc, src, dst)
    h1c = _layer1(xp, agg1c, deg2, W_self1, W_neigh1, b1.reshape(1, -1))
    (agg2c,) = _make_sc_agg(4, False)(h1c, src, dst)
    scores = _layer2_head(h1c, agg2c, deg2, W_self2, W_neigh2,
                          b2.reshape(1, -1), W_fc1, b_fc1.reshape(1, -1),
                          W_fc2, b_fc2.reshape(1, 1))
    return scores[:N]
